# baseline (speedup 1.0000x reference)
import functools

import jax
import jax.numpy as jnp
from jax import lax
from jax.experimental import pallas as pl
from jax.experimental.pallas import tpu as pltpu
from jax.experimental.pallas import tpu_sc as plsc

B = 16384
D = 16
H1 = 64
H2 = 32


@functools.cache
def _sc_gather(nb):
    info = plsc.get_sparse_core_info()
    nw = info.num_cores * info.num_subcores
    bpw = nb // nw
    mesh = plsc.VectorSubcoreMesh(core_axis_name="c", subcore_axis_name="s")

    @functools.partial(
        pl.kernel,
        out_type=(
            jax.ShapeDtypeStruct((D, nb), jnp.float32),
            jax.ShapeDtypeStruct((D, nb), jnp.float32),
        ),
        mesh=mesh,
        compiler_params=pltpu.CompilerParams(use_tc_tiling_on_sc=True,
                                             needs_layout_passes=False),
        scratch_types=[
            pltpu.VMEM((bpw,), jnp.int32),
            pltpu.VMEM((bpw,), jnp.int32),
            pltpu.VMEM((D, 16 * 128), jnp.float32),
            pltpu.VMEM((D, 16 * 128), jnp.float32),
            pltpu.VMEM((D, bpw), jnp.float32),
            pltpu.VMEM((D, bpw), jnp.float32),
            pltpu.SemaphoreType.DMA,
            pltpu.SemaphoreType.DMA,
        ],
    )
    def gather_kernel(uid_hbm, iid_hbm, utab_hbm, itab_hbm, uout_hbm, iout_hbm,
                      uidx_v, iidx_v, ubuf_v, ibuf_v, urows_v, irows_v,
                      usem, isem):
        wid = lax.axis_index("s") * info.num_cores + lax.axis_index("c")
        base = wid * bpw
        pltpu.sync_copy(uid_hbm.at[pl.ds(base, bpw)], uidx_v)
        pltpu.sync_copy(iid_hbm.at[pl.ds(base, bpw)], iidx_v)
        lanes = lax.iota(jnp.int32, 16)

        def group(m, _):
            uvec = uidx_v[pl.ds(m * 16, 16)]
            ivec = iidx_v[pl.ds(m * 16, 16)]
            copies = []
            for j in range(16):
                ru = uvec[j]
                ri = ivec[j]
                qu = pl.multiple_of(
                    lax.shift_left(lax.shift_right_logical(ru, 7), 7), 128)
                qi = pl.multiple_of(
                    lax.shift_left(lax.shift_right_logical(ri, 7), 7), 128)
                copies.append(pltpu.async_copy(
                    utab_hbm.at[:, pl.ds(qu, 128)],
                    ubuf_v.at[:, pl.ds(j * 128, 128)], usem))
                copies.append(pltpu.async_copy(
                    itab_hbm.at[:, pl.ds(qi, 128)],
                    ibuf_v.at[:, pl.ds(j * 128, 128)], isem))
            for cp in copies:
                cp.wait()
            for j in range(16):
                ru = uvec[j]
                ri = ivec[j]
                lu = lanes * 0 + (j * 128 + lax.bitwise_and(ru, 127))
                li = lanes * 0 + (j * 128 + lax.bitwise_and(ri, 127))
                col = lanes * 0 + (m * 16 + j)
                uval = plsc.load_gather(ubuf_v, [lanes, lu])
                plsc.store_scatter(urows_v, [lanes, col], uval)
                ival = plsc.load_gather(ibuf_v, [lanes, li])
                plsc.store_scatter(irows_v, [lanes, col], ival)
            return 0

        lax.fori_loop(0, bpw // 16, group, 0)
        aligned_base = pl.multiple_of(base, 128)
        pltpu.sync_copy(urows_v, uout_hbm.at[:, pl.ds(aligned_base, bpw)])
        pltpu.sync_copy(irows_v, iout_hbm.at[:, pl.ds(aligned_base, bpw)])

    return gather_kernel


_BB = 16384


def _tc_body(u_ref, i_ref, w1ut_ref, w1it_ref, b1_ref, w2t_ref, b2_ref,
             w3_ref, c0_ref, out_ref):
    uT = u_ref[...]
    iT = i_ref[...]
    inter = jnp.sum(uT * iT, axis=0)
    h1 = jnp.dot(w1ut_ref[...], uT, preferred_element_type=jnp.float32)
    h1 = h1 + jnp.dot(w1it_ref[...], iT, preferred_element_type=jnp.float32)
    h1 = jnp.maximum(h1 + b1_ref[...], 0.0)
    h2 = jnp.dot(w2t_ref[...], h1, preferred_element_type=jnp.float32)
    h2 = jnp.maximum(h2 + b2_ref[...], 0.0)
    deep = jnp.sum(h2 * w3_ref[...], axis=0)
    out_ref[...] = inter + deep + c0_ref[0]


def _tc_mlp(uT, iT, w1ut, w1it, b1col, w2t, b2col, w3col, c0):
    nb = uT.shape[1]
    rep = lambda shape: pl.BlockSpec(shape, lambda i: (0,) * len(shape))
    return pl.pallas_call(
        _tc_body,
        grid=(nb // _BB,),
        in_specs=[
            pl.BlockSpec((D, _BB), lambda i: (0, i)),
            pl.BlockSpec((D, _BB), lambda i: (0, i)),
            rep((H1, D)),
            rep((H1, D)),
            rep((H1, 1)),
            rep((H2, H1)),
            rep((H2, 1)),
            rep((H2, 1)),
            pl.BlockSpec(memory_space=pltpu.SMEM),
        ],
        out_specs=pl.BlockSpec((_BB,), lambda i: (i,)),
        out_shape=jax.ShapeDtypeStruct((nb,), jnp.float32),
    )(uT, iT, w1ut, w1it, b1col, w2t, b2col, w3col, c0)


_NSPLIT = 1


def kernel(user_id, item_id, user_table, item_table, fm_bias, W1, b1, W2, b2,
           W3, b3):
    uid = user_id.astype(jnp.int32)
    iid = item_id.astype(jnp.int32)
    utabT = user_table.T
    itabT = item_table.T
    c0 = fm_bias + b3
    W1t = W1.T
    nb = B // _NSPLIT
    gather = _sc_gather(nb)
    outs = []
    for k in range(_NSPLIT):
        uT, iT = gather(lax.dynamic_slice_in_dim(uid, k * nb, nb),
                        lax.dynamic_slice_in_dim(iid, k * nb, nb),
                        utabT, itabT)
        outs.append(_tc_mlp(uT, iT, W1t[:, :D], W1t[:, D:],
                            b1.reshape(H1, 1), W2.T, b2.reshape(H2, 1), W3,
                            c0))
    return outs[0] if _NSPLIT == 1 else jnp.concatenate(outs)


# speedup vs baseline: 1.1512x; 1.1512x over previous
import functools

import jax
import jax.numpy as jnp
from jax import lax
from jax.experimental import pallas as pl
from jax.experimental.pallas import tpu as pltpu
from jax.experimental.pallas import tpu_sc as plsc

B = 16384
D = 16
H1 = 64
H2 = 32


@functools.cache
def _sc_gather(nb):
    info = plsc.get_sparse_core_info()
    nw = info.num_cores * info.num_subcores
    bpw = nb // nw
    mesh = plsc.VectorSubcoreMesh(core_axis_name="c", subcore_axis_name="s")

    @functools.partial(
        pl.kernel,
        out_type=(
            jax.ShapeDtypeStruct((D, nb), jnp.float32),
            jax.ShapeDtypeStruct((D, nb), jnp.float32),
        ),
        mesh=mesh,
        compiler_params=pltpu.CompilerParams(use_tc_tiling_on_sc=True,
                                             needs_layout_passes=False),
        scratch_types=[
            pltpu.VMEM((bpw,), jnp.int32),
            pltpu.VMEM((bpw,), jnp.int32),
            pltpu.VMEM((D, 16 * 128), jnp.float32),
            pltpu.VMEM((D, 16 * 128), jnp.float32),
            pltpu.VMEM((D, bpw), jnp.float32),
            pltpu.VMEM((D, bpw), jnp.float32),
            pltpu.SemaphoreType.DMA,
            pltpu.SemaphoreType.DMA,
            pltpu.SemaphoreType.DMA,
            pltpu.SemaphoreType.DMA,
        ],
    )
    def gather_kernel(uid_hbm, iid_hbm, utab_hbm, itab_hbm, uout_hbm, iout_hbm,
                      uidx_v, iidx_v, ubuf_v, ibuf_v, urows_v, irows_v,
                      usem_a, isem_a, usem_b, isem_b):
        wid = lax.axis_index("s") * info.num_cores + lax.axis_index("c")
        base = wid * bpw
        pltpu.sync_copy(uid_hbm.at[pl.ds(base, bpw)], uidx_v)
        pltpu.sync_copy(iid_hbm.at[pl.ds(base, bpw)], iidx_v)
        lanes = lax.iota(jnp.int32, 16)
        ngrp = bpw // 8

        def tile_col(r):
            return pl.multiple_of(
                lax.shift_left(lax.shift_right_logical(r, 7), 7), 128)

        def issue(uvec, ivec, half, slot, usem, isem):
            for j in range(8):
                ru = uvec[half * 8 + j]
                ri = ivec[half * 8 + j]
                dst = pl.ds(slot * 1024 + j * 128, 128)
                pltpu.async_copy(utab_hbm.at[:, pl.ds(tile_col(ru), 128)],
                                 ubuf_v.at[:, dst], usem)
                pltpu.async_copy(itab_hbm.at[:, pl.ds(tile_col(ri), 128)],
                                 ibuf_v.at[:, dst], isem)

        def drain(slot, usem, isem):
            sl = pl.ds(slot * 1024, 1024)
            pltpu.make_async_copy(utab_hbm.at[:, pl.ds(0, 1024)],
                                  ubuf_v.at[:, sl], usem).wait()
            pltpu.make_async_copy(itab_hbm.at[:, pl.ds(0, 1024)],
                                  ibuf_v.at[:, sl], isem).wait()

        def extract(g, uvec, ivec, half, slot):
            for j in range(8):
                ru = uvec[half * 8 + j]
                ri = ivec[half * 8 + j]
                off = slot * 1024 + j * 128
                lu = lanes * 0 + (off + lax.bitwise_and(ru, 127))
                li = lanes * 0 + (off + lax.bitwise_and(ri, 127))
                col = lanes * 0 + (g * 8 + j)
                uval = plsc.load_gather(ubuf_v, [lanes, lu])
                plsc.store_scatter(urows_v, [lanes, col], uval)
                ival = plsc.load_gather(ibuf_v, [lanes, li])
                plsc.store_scatter(irows_v, [lanes, col], ival)

        def vecs_for_pair(mm):
            uvec = uidx_v[pl.ds(mm * 16, 16)]
            ivec = iidx_v[pl.ds(mm * 16, 16)]
            return uvec, ivec

        uvec0, ivec0 = vecs_for_pair(0)
        issue(uvec0, ivec0, 0, 0, usem_a, isem_a)
        issue(uvec0, ivec0, 1, 1, usem_b, isem_b)

        def pipe(mm, _):
            uvec, ivec = vecs_for_pair(mm)
            nxt = jnp.minimum(mm + 1, ngrp // 2 - 1)
            uvn, ivn = vecs_for_pair(nxt)
            drain(0, usem_a, isem_a)
            extract(2 * mm, uvec, ivec, 0, 0)
            issue(uvn, ivn, 0, 0, usem_a, isem_a)
            drain(1, usem_b, isem_b)
            extract(2 * mm + 1, uvec, ivec, 1, 1)
            issue(uvn, ivn, 1, 1, usem_b, isem_b)
            return 0

        lax.fori_loop(0, ngrp // 2, pipe, 0)
        drain(0, usem_a, isem_a)
        drain(1, usem_b, isem_b)
        aligned_base = pl.multiple_of(base, 128)
        pltpu.sync_copy(urows_v, uout_hbm.at[:, pl.ds(aligned_base, bpw)])
        pltpu.sync_copy(irows_v, iout_hbm.at[:, pl.ds(aligned_base, bpw)])

    return gather_kernel


_BB = 8192


def _tc_body(u_ref, i_ref, w1ut_ref, w1it_ref, b1_ref, w2t_ref, b2_ref,
             w3_ref, c0_ref, out_ref):
    uT = u_ref[...]
    iT = i_ref[...]
    inter = jnp.sum(uT * iT, axis=0)
    h1 = jnp.dot(w1ut_ref[...], uT, preferred_element_type=jnp.float32)
    h1 = h1 + jnp.dot(w1it_ref[...], iT, preferred_element_type=jnp.float32)
    h1 = jnp.maximum(h1 + b1_ref[...], 0.0)
    h2 = jnp.dot(w2t_ref[...], h1, preferred_element_type=jnp.float32)
    h2 = jnp.maximum(h2 + b2_ref[...], 0.0)
    deep = jnp.sum(h2 * w3_ref[...], axis=0)
    out_ref[...] = inter + deep + c0_ref[0]


def _tc_mlp(uT, iT, w1ut, w1it, b1col, w2t, b2col, w3col, c0):
    nb = uT.shape[1]
    rep = lambda shape: pl.BlockSpec(shape, lambda i: (0,) * len(shape))
    return pl.pallas_call(
        _tc_body,
        grid=(nb // _BB,),
        in_specs=[
            pl.BlockSpec((D, _BB), lambda i: (0, i)),
            pl.BlockSpec((D, _BB), lambda i: (0, i)),
            rep((H1, D)),
            rep((H1, D)),
            rep((H1, 1)),
            rep((H2, H1)),
            rep((H2, 1)),
            rep((H2, 1)),
            pl.BlockSpec(memory_space=pltpu.SMEM),
        ],
        out_specs=pl.BlockSpec((_BB,), lambda i: (i,)),
        out_shape=jax.ShapeDtypeStruct((nb,), jnp.float32),
    )(uT, iT, w1ut, w1it, b1col, w2t, b2col, w3col, c0)


_NSPLIT = 1


def kernel(user_id, item_id, user_table, item_table, fm_bias, W1, b1, W2, b2,
           W3, b3):
    uid = user_id.astype(jnp.int32)
    iid = item_id.astype(jnp.int32)
    utabT = user_table.T
    itabT = item_table.T
    c0 = fm_bias + b3
    W1t = W1.T
    nb = B // _NSPLIT
    gather = _sc_gather(nb)
    outs = []
    for k in range(_NSPLIT):
        uT, iT = gather(lax.dynamic_slice_in_dim(uid, k * nb, nb),
                        lax.dynamic_slice_in_dim(iid, k * nb, nb),
                        utabT, itabT)
        outs.append(_tc_mlp(uT, iT, W1t[:, :D], W1t[:, D:],
                            b1.reshape(H1, 1), W2.T, b2.reshape(H2, 1), W3,
                            c0))
    return outs[0] if _NSPLIT == 1 else jnp.concatenate(outs)
